# Initial kernel scaffold; baseline (speedup 1.0000x reference)
import jax
import jax.numpy as jnp
from jax.experimental import pallas as pl


def kernel(x, edge_index, edge_attr, batch, W1a, b1a, W1b, b1b, We1, be1, W2a, b2a, W2b, b2b, We2, be2, Wp1, bp1, Wp2, bp2, g1, beta1, g2, beta2):
    raise NotImplementedError("write your pallas kernel here")


# speedup vs baseline: 2.0964x; 2.0964x over previous
import functools

import jax
import jax.numpy as jnp
from jax import lax
from jax.experimental import pallas as pl
from jax.experimental.pallas import tpu as pltpu
from jax.experimental.pallas import tpu_sc as plsc

N = 10000
E = 160000
D = 256
DE = 16
G = 64

NC = 2
NS = 16
HALF = D // NC
EPT = E // NS
K = 80
NCH = EPT // K
WBT = 10
RPT = N // WBT


def _elin_body(ea_ref, we_ref, be_ref, out_ref):
    c = pl.program_id(0)
    acc = jnp.dot(ea_ref[...], we_ref[...], preferred_element_type=jnp.float32)
    out_ref[0] = acc + be_ref[pl.ds(c, 1)]


def _elin(edge_attr, We, be2d):
    BE = 2000
    return pl.pallas_call(
        _elin_body,
        grid=(NC, E // BE),
        in_specs=[
            pl.BlockSpec((BE, DE), lambda c, i: (i, 0)),
            pl.BlockSpec((DE, HALF), lambda c, i: (0, c)),
            pl.BlockSpec((NC, HALF), lambda c, i: (0, 0)),
        ],
        out_specs=pl.BlockSpec((1, BE, HALF), lambda c, i: (c, i, 0)),
        out_shape=jax.ShapeDtypeStruct((NC, E, HALF), jnp.float32),
    )(edge_attr, We, be2d)


def _msg_body(xcat, elin2d, src4, dst3, zrows, out, idx_v, dst_v, gbuf,
              ebuf, agg, gsem, esem):
    c = lax.axis_index("c")
    s = lax.axis_index("s")

    @pl.when(s < WBT)
    def _():
        pltpu.sync_copy(zrows, agg.at[pl.ds(s * RPT, RPT)])

    plsc.subcore_barrier()

    ebase = c * E + s * EPT

    def chunk(j, _):
        pltpu.sync_copy(src4.at[c, s, j], idx_v.at[0])
        pltpu.sync_copy(dst3.at[s, j], dst_v.at[0])
        gd = pltpu.async_copy(xcat.at[idx_v.at[0]], gbuf, gsem)
        ed = pltpu.async_copy(elin2d.at[pl.ds(ebase + j * K, K)], ebuf, esem)
        gd.wait()
        ed.wait()

        def row(r, _):
            for q in range(HALF // 16):
                sl = pl.ds(q * 16, 16)
                gbuf[r, sl] = jnp.maximum(gbuf[r, sl] + ebuf[r, sl], 0.0)
            return 0

        lax.fori_loop(0, K, row, 0)
        pltpu.sync_copy(gbuf, agg.at[dst_v.at[0]], add=True)
        return 0

    lax.fori_loop(0, NCH, chunk, 0)
    plsc.subcore_barrier()

    @pl.when(s < WBT)
    def _():
        pltpu.sync_copy(agg.at[pl.ds(s * RPT, RPT)],
                        out.at[c, pl.ds(s * RPT, RPT)])


def _msg(xcat, elin2d, src4, dst3, zrows):
    mesh = plsc.VectorSubcoreMesh(core_axis_name="c", subcore_axis_name="s")
    kern = pl.kernel(
        _msg_body,
        mesh=mesh,
        out_type=jax.ShapeDtypeStruct((NC, N, HALF), jnp.float32),
        scratch_types=[
            pltpu.VMEM((2, K), jnp.int32),
            pltpu.VMEM((2, K), jnp.int32),
            pltpu.VMEM((K, HALF), jnp.float32),
            pltpu.VMEM((K, HALF), jnp.float32),
            pltpu.VMEM_SHARED((N, HALF), jnp.float32),
            pltpu.SemaphoreType.DMA,
            pltpu.SemaphoreType.DMA,
        ],
    )
    return kern(xcat, elin2d, src4, dst3, zrows)


def _stageb1_body(x_ref, agg_ref, wa, ba, wb, bb, wp, bp, gs, beta,
                  h_ref, hh_ref):
    agg = jnp.concatenate([agg_ref[0], agg_ref[1]], axis=1)
    hin = x_ref[...] + agg
    t = jnp.maximum(jnp.dot(hin, wa[...], preferred_element_type=jnp.float32)
                    + ba[...], 0.0)
    u = jnp.maximum(jnp.dot(t, wb[...], preferred_element_type=jnp.float32)
                    + bb[...], 0.0)
    v = u * gs[...] + beta[...]
    res = jnp.dot(x_ref[...], wp[...], preferred_element_type=jnp.float32) + bp[...]
    h = v + res
    h_ref[...] = h
    hh_ref[0] = h[:, :HALF]
    hh_ref[1] = h[:, HALF:]


def _stageb1(x, agg, Wa, ba, Wb, bb, Wp, bp, gs, beta):
    NB = 1000
    full = lambda i: (0, 0)
    return pl.pallas_call(
        _stageb1_body,
        grid=(N // NB,),
        in_specs=[
            pl.BlockSpec((NB, D), lambda i: (i, 0)),
            pl.BlockSpec((NC, NB, HALF), lambda i: (0, i, 0)),
            pl.BlockSpec((D, D), full),
            pl.BlockSpec((1, D), full),
            pl.BlockSpec((D, D), full),
            pl.BlockSpec((1, D), full),
            pl.BlockSpec((D, D), full),
            pl.BlockSpec((1, D), full),
            pl.BlockSpec((1, D), full),
            pl.BlockSpec((1, D), full),
        ],
        out_specs=[
            pl.BlockSpec((NB, D), lambda i: (i, 0)),
            pl.BlockSpec((NC, NB, HALF), lambda i: (0, i, 0)),
        ],
        out_shape=[
            jax.ShapeDtypeStruct((N, D), jnp.float32),
            jax.ShapeDtypeStruct((NC, N, HALF), jnp.float32),
        ],
    )(x, agg, Wa, ba, Wb, bb, Wp, bp, gs, beta)


def _stageb2_body(x_ref, agg_ref, wa, ba, wb, bb, wp, bp, gs, beta, batch_ref,
                  out_ref):
    i = pl.program_id(0)
    agg = jnp.concatenate([agg_ref[0], agg_ref[1]], axis=1)
    hin = x_ref[...] + agg
    t = jnp.maximum(jnp.dot(hin, wa[...], preferred_element_type=jnp.float32)
                    + ba[...], 0.0)
    u = jnp.maximum(jnp.dot(t, wb[...], preferred_element_type=jnp.float32)
                    + bb[...], 0.0)
    v = u * gs[...] + beta[...]
    res = jnp.dot(x_ref[...], wp[...], preferred_element_type=jnp.float32) + bp[...]
    h2 = v + res
    nb = h2.shape[0]
    bmat = jnp.broadcast_to(batch_ref[...], (nb, 128))
    gids = lax.broadcasted_iota(jnp.int32, (nb, 128), 1)
    onehot = jnp.where(bmat == gids, 1.0, 0.0).astype(jnp.float32)
    part = lax.dot_general(onehot, h2, (((0,), (0,)), ((), ())),
                           preferred_element_type=jnp.float32)

    @pl.when(i == 0)
    def _():
        out_ref[...] = part

    @pl.when(i != 0)
    def _():
        out_ref[...] = out_ref[...] + part


def _stageb2(x, agg, Wa, ba, Wb, bb, Wp, bp, gs, beta, batch2d):
    NB = 1000
    full = lambda i: (0, 0)
    return pl.pallas_call(
        _stageb2_body,
        grid=(N // NB,),
        in_specs=[
            pl.BlockSpec((NB, D), lambda i: (i, 0)),
            pl.BlockSpec((NC, NB, HALF), lambda i: (0, i, 0)),
            pl.BlockSpec((D, D), full),
            pl.BlockSpec((1, D), full),
            pl.BlockSpec((D, D), full),
            pl.BlockSpec((1, D), full),
            pl.BlockSpec((D, D), full),
            pl.BlockSpec((1, D), full),
            pl.BlockSpec((1, D), full),
            pl.BlockSpec((1, D), full),
            pl.BlockSpec((NB, 1), lambda i: (i, 0)),
        ],
        out_specs=pl.BlockSpec((128, D), full),
        out_shape=jax.ShapeDtypeStruct((128, D), jnp.float32),
    )(x, agg, Wa, ba, Wb, bb, Wp, bp, gs, beta, batch2d)


def kernel(x, edge_index, edge_attr, batch,
           W1a, b1a, W1b, b1b, We1, be1,
           W2a, b2a, W2b, b2b, We2, be2,
           Wp1, bp1, Wp2, bp2, g1, beta1, g2, beta2):
    src = edge_index[0]
    dst = edge_index[1]

    src3 = src.reshape(NS, NCH, K)
    src4 = jnp.stack([src3, src3 + N])
    dst3 = dst.reshape(NS, NCH, K)
    zrows = jnp.zeros((RPT, HALF), jnp.float32)

    bn_scale = 1.0 / jnp.sqrt(1.0 + 1e-5)
    gs1 = (g1 * bn_scale).reshape(1, D)
    gs2 = (g2 * bn_scale).reshape(1, D)

    elin1 = _elin(edge_attr, We1, be1.reshape(NC, HALF)).reshape(NC * E, HALF)
    elin2 = _elin(edge_attr, We2, be2.reshape(NC, HALF)).reshape(NC * E, HALF)

    xcat = jnp.concatenate([x[:, :HALF], x[:, HALF:]], axis=0)
    agg1 = _msg(xcat, elin1, src4, dst3, zrows)

    h, hh = _stageb1(x, agg1, W1a, b1a.reshape(1, D), W1b, b1b.reshape(1, D),
                     Wp1, bp1.reshape(1, D), gs1, beta1.reshape(1, D))

    agg2 = _msg(hh.reshape(NC * N, HALF), elin2, src4, dst3, zrows)

    out128 = _stageb2(h, agg2, W2a, b2a.reshape(1, D), W2b, b2b.reshape(1, D),
                      Wp2, bp2.reshape(1, D), gs2, beta2.reshape(1, D),
                      batch.reshape(N, 1))
    return out128[:G]


# speedup vs baseline: 3.2117x; 1.5320x over previous
import functools

import jax
import jax.numpy as jnp
from jax import lax
from jax.experimental import pallas as pl
from jax.experimental.pallas import tpu as pltpu
from jax.experimental.pallas import tpu_sc as plsc

N = 10000
E = 160000
D = 256
DE = 16
G = 64

NC = 2
NS = 16
HALF = D // NC
EPT = E // NS
K = 80
NCH = EPT // K
WBT = 10
RPT = N // WBT


def _elin_body(ea_ref, we_ref, be_ref, out_ref):
    c = pl.program_id(0)
    acc = jnp.dot(ea_ref[...], we_ref[...], preferred_element_type=jnp.float32)
    out_ref[0] = acc + be_ref[pl.ds(c, 1)]


def _elin(edge_attr, We, be2d):
    BE = 2000
    return pl.pallas_call(
        _elin_body,
        grid=(NC, E // BE),
        in_specs=[
            pl.BlockSpec((BE, DE), lambda c, i: (i, 0)),
            pl.BlockSpec((DE, HALF), lambda c, i: (0, c)),
            pl.BlockSpec((NC, HALF), lambda c, i: (0, 0)),
        ],
        out_specs=pl.BlockSpec((1, BE, HALF), lambda c, i: (c, i, 0)),
        out_shape=jax.ShapeDtypeStruct((NC, E, HALF), jnp.float32),
    )(edge_attr, We, be2d)


def _msg_body(xcat, elin2d, comb, zrows, out,
              cidx0, cidx1, gbuf0, gbuf1, ebuf0, ebuf1, agg,
              gsem0, gsem1, esem0, esem1, ssem0, ssem1, isem):
    c = lax.axis_index("c")
    s = lax.axis_index("s")

    @pl.when(s < WBT)
    def _():
        pltpu.sync_copy(zrows, agg.at[pl.ds(s * RPT, RPT)])

    plsc.subcore_barrier()

    ebase = c * E + s * EPT

    def elin_rows(j):
        return elin2d.at[pl.ds(ebase + j * K, K)]

    def compute(gbuf, ebuf):
        def row(r, _):
            for q in range(HALF // 16):
                sl = pl.ds(q * 16, 16)
                gbuf[r, sl] = jnp.maximum(gbuf[r, sl] + ebuf[r, sl], 0.0)
            return 0

        lax.fori_loop(0, K, row, 0)

    def step(j, cur, nxt):
        (cidx_c, gbuf_c, ebuf_c, gsem_c, esem_c, ssem_c) = cur
        (cidx_n, gbuf_n, ebuf_n, gsem_n, esem_n, ssem_n) = nxt

        @pl.when(j >= 1)
        def _():
            pltpu.make_async_copy(gbuf_n, agg.at[cidx_n.at[1]], ssem_n).wait()

        @pl.when(j + 1 < NCH)
        def _():
            pltpu.async_copy(comb.at[c, s, j + 1], cidx_n, isem)

        pltpu.make_async_copy(xcat.at[cidx_c.at[0]], gbuf_c, gsem_c).wait()
        pltpu.make_async_copy(elin_rows(j), ebuf_c, esem_c).wait()

        @pl.when(j + 1 < NCH)
        def _():
            pltpu.make_async_copy(comb.at[c, s, 0], cidx_n, isem).wait()
            pltpu.async_copy(xcat.at[cidx_n.at[0]], gbuf_n, gsem_n)
            pltpu.async_copy(elin_rows(j + 1), ebuf_n, esem_n)

        compute(gbuf_c, ebuf_c)
        pltpu.async_copy(gbuf_c, agg.at[cidx_c.at[1]], ssem_c, add=True)

    slot0 = (cidx0, gbuf0, ebuf0, gsem0, esem0, ssem0)
    slot1 = (cidx1, gbuf1, ebuf1, gsem1, esem1, ssem1)

    pltpu.sync_copy(comb.at[c, s, 0], cidx0)
    pltpu.async_copy(xcat.at[cidx0.at[0]], gbuf0, gsem0)
    pltpu.async_copy(elin_rows(0), ebuf0, esem0)

    def pair(t, _):
        step(2 * t, slot0, slot1)
        step(2 * t + 1, slot1, slot0)
        return 0

    lax.fori_loop(0, NCH // 2, pair, 0)
    if NCH % 2:
        step(NCH - 1, slot0, slot1)
        pltpu.make_async_copy(gbuf0, agg.at[cidx0.at[1]], ssem0).wait()
    else:
        pltpu.make_async_copy(gbuf1, agg.at[cidx1.at[1]], ssem1).wait()
    plsc.subcore_barrier()

    @pl.when(s < WBT)
    def _():
        pltpu.sync_copy(agg.at[pl.ds(s * RPT, RPT)],
                        out.at[c, pl.ds(s * RPT, RPT)])


def _msg(xcat, elin2d, comb, zrows):
    mesh = plsc.VectorSubcoreMesh(core_axis_name="c", subcore_axis_name="s")
    kern = pl.kernel(
        _msg_body,
        mesh=mesh,
        out_type=jax.ShapeDtypeStruct((NC, N, HALF), jnp.float32),
        scratch_types=[
            pltpu.VMEM((2, K), jnp.int32),
            pltpu.VMEM((2, K), jnp.int32),
            pltpu.VMEM((K, HALF), jnp.float32),
            pltpu.VMEM((K, HALF), jnp.float32),
            pltpu.VMEM((K, HALF), jnp.float32),
            pltpu.VMEM((K, HALF), jnp.float32),
            pltpu.VMEM_SHARED((N, HALF), jnp.float32),
        ] + [pltpu.SemaphoreType.DMA] * 7,
    )
    return kern(xcat, elin2d, comb, zrows)


def _stageb1_body(x_ref, agg_ref, wa, ba, wb, bb, wp, bp, gs, beta,
                  h_ref, hh_ref):
    agg = jnp.concatenate([agg_ref[0], agg_ref[1]], axis=1)
    hin = x_ref[...] + agg
    t = jnp.maximum(jnp.dot(hin, wa[...], preferred_element_type=jnp.float32)
                    + ba[...], 0.0)
    u = jnp.maximum(jnp.dot(t, wb[...], preferred_element_type=jnp.float32)
                    + bb[...], 0.0)
    v = u * gs[...] + beta[...]
    res = jnp.dot(x_ref[...], wp[...], preferred_element_type=jnp.float32) + bp[...]
    h = v + res
    h_ref[...] = h
    hh_ref[0] = h[:, :HALF]
    hh_ref[1] = h[:, HALF:]


def _stageb1(x, agg, Wa, ba, Wb, bb, Wp, bp, gs, beta):
    NB = 1000
    full = lambda i: (0, 0)
    return pl.pallas_call(
        _stageb1_body,
        grid=(N // NB,),
        in_specs=[
            pl.BlockSpec((NB, D), lambda i: (i, 0)),
            pl.BlockSpec((NC, NB, HALF), lambda i: (0, i, 0)),
            pl.BlockSpec((D, D), full),
            pl.BlockSpec((1, D), full),
            pl.BlockSpec((D, D), full),
            pl.BlockSpec((1, D), full),
            pl.BlockSpec((D, D), full),
            pl.BlockSpec((1, D), full),
            pl.BlockSpec((1, D), full),
            pl.BlockSpec((1, D), full),
        ],
        out_specs=[
            pl.BlockSpec((NB, D), lambda i: (i, 0)),
            pl.BlockSpec((NC, NB, HALF), lambda i: (0, i, 0)),
        ],
        out_shape=[
            jax.ShapeDtypeStruct((N, D), jnp.float32),
            jax.ShapeDtypeStruct((NC, N, HALF), jnp.float32),
        ],
    )(x, agg, Wa, ba, Wb, bb, Wp, bp, gs, beta)


def _stageb2_body(x_ref, agg_ref, wa, ba, wb, bb, wp, bp, gs, beta, batch_ref,
                  out_ref):
    i = pl.program_id(0)
    agg = jnp.concatenate([agg_ref[0], agg_ref[1]], axis=1)
    hin = x_ref[...] + agg
    t = jnp.maximum(jnp.dot(hin, wa[...], preferred_element_type=jnp.float32)
                    + ba[...], 0.0)
    u = jnp.maximum(jnp.dot(t, wb[...], preferred_element_type=jnp.float32)
                    + bb[...], 0.0)
    v = u * gs[...] + beta[...]
    res = jnp.dot(x_ref[...], wp[...], preferred_element_type=jnp.float32) + bp[...]
    h2 = v + res
    nb = h2.shape[0]
    bmat = jnp.broadcast_to(batch_ref[...], (nb, 128))
    gids = lax.broadcasted_iota(jnp.int32, (nb, 128), 1)
    onehot = jnp.where(bmat == gids, 1.0, 0.0).astype(jnp.float32)
    part = lax.dot_general(onehot, h2, (((0,), (0,)), ((), ())),
                           preferred_element_type=jnp.float32)

    @pl.when(i == 0)
    def _():
        out_ref[...] = part

    @pl.when(i != 0)
    def _():
        out_ref[...] = out_ref[...] + part


def _stageb2(x, agg, Wa, ba, Wb, bb, Wp, bp, gs, beta, batch2d):
    NB = 1000
    full = lambda i: (0, 0)
    return pl.pallas_call(
        _stageb2_body,
        grid=(N // NB,),
        in_specs=[
            pl.BlockSpec((NB, D), lambda i: (i, 0)),
            pl.BlockSpec((NC, NB, HALF), lambda i: (0, i, 0)),
            pl.BlockSpec((D, D), full),
            pl.BlockSpec((1, D), full),
            pl.BlockSpec((D, D), full),
            pl.BlockSpec((1, D), full),
            pl.BlockSpec((D, D), full),
            pl.BlockSpec((1, D), full),
            pl.BlockSpec((1, D), full),
            pl.BlockSpec((1, D), full),
            pl.BlockSpec((NB, 1), lambda i: (i, 0)),
        ],
        out_specs=pl.BlockSpec((128, D), full),
        out_shape=jax.ShapeDtypeStruct((128, D), jnp.float32),
    )(x, agg, Wa, ba, Wb, bb, Wp, bp, gs, beta, batch2d)


def kernel(x, edge_index, edge_attr, batch,
           W1a, b1a, W1b, b1b, We1, be1,
           W2a, b2a, W2b, b2b, We2, be2,
           Wp1, bp1, Wp2, bp2, g1, beta1, g2, beta2):
    src = edge_index[0]
    dst = edge_index[1]

    src3 = src.reshape(NS, NCH, K)
    dst3 = dst.reshape(NS, NCH, K)
    comb = jnp.stack([jnp.stack([src3, dst3], axis=2),
                      jnp.stack([src3 + N, dst3], axis=2)])
    zrows = jnp.zeros((RPT, HALF), jnp.float32)

    bn_scale = 1.0 / jnp.sqrt(1.0 + 1e-5)
    gs1 = (g1 * bn_scale).reshape(1, D)
    gs2 = (g2 * bn_scale).reshape(1, D)

    elin1 = _elin(edge_attr, We1, be1.reshape(NC, HALF)).reshape(NC * E, HALF)
    elin2 = _elin(edge_attr, We2, be2.reshape(NC, HALF)).reshape(NC * E, HALF)

    xcat = jnp.concatenate([x[:, :HALF], x[:, HALF:]], axis=0)
    agg1 = _msg(xcat, elin1, comb, zrows)

    h, hh = _stageb1(x, agg1, W1a, b1a.reshape(1, D), W1b, b1b.reshape(1, D),
                     Wp1, bp1.reshape(1, D), gs1, beta1.reshape(1, D))

    agg2 = _msg(hh.reshape(NC * N, HALF), elin2, comb, zrows)

    out128 = _stageb2(h, agg2, W2a, b2a.reshape(1, D), W2b, b2b.reshape(1, D),
                      Wp2, bp2.reshape(1, D), gs2, beta2.reshape(1, D),
                      batch.reshape(N, 1))
    return out128[:G]
